# Initial kernel scaffold; baseline (speedup 1.0000x reference)
import jax
import jax.numpy as jnp
from jax.experimental import pallas as pl


def kernel(data, lengths, embed_init):
    raise NotImplementedError("write your pallas kernel here")


# speedup vs baseline: 2.2840x; 2.2840x over previous
import functools

import jax
import jax.numpy as jnp
from jax import lax
from jax.experimental import pallas as pl
from jax.experimental.pallas import tpu as pltpu
from jax.experimental.pallas import tpu_sc as plsc

MAX_LEN = 2048
BATCH = 16
NUM_AA = 21
ROW = BATCH * NUM_AA
NUM_CORES = 2
NUM_SUBCORES = 16
NUM_WORKERS = NUM_CORES * NUM_SUBCORES
T_PER_W = MAX_LEN // NUM_WORKERS

_mesh = plsc.VectorSubcoreMesh(core_axis_name="c", subcore_axis_name="s")


@functools.partial(
    pl.kernel,
    mesh=_mesh,
    out_type=jax.ShapeDtypeStruct((MAX_LEN * ROW,), jnp.float32),
    scratch_types=[
        pltpu.VMEM((T_PER_W, BATCH), jnp.int32),
        pltpu.VMEM((BATCH,), jnp.int32),
        pltpu.VMEM((T_PER_W * ROW,), jnp.float32),
    ],
    compiler_params=pltpu.CompilerParams(needs_layout_passes=False),
)
def _onehot_sc(data_hbm, len_hbm, out_hbm, data_v, len_v, out_v):
    wid = lax.axis_index("s") * NUM_CORES + lax.axis_index("c")
    t0 = wid * T_PER_W
    pltpu.sync_copy(data_hbm.at[pl.ds(t0, T_PER_W)], data_v)
    pltpu.sync_copy(len_hbm, len_v)

    lanes = lax.iota(jnp.int32, 16)
    lens = len_v[...]
    zero16 = jnp.zeros((16,), jnp.float32)
    one16 = jnp.ones((16,), jnp.float32)

    def body(t, carry):
        base = t * ROW
        for j in range(ROW // 16):
            out_v[pl.ds(base + j * 16, 16)] = zero16
        row = data_v[t]
        idx = base + lanes * NUM_AA + row
        mask = (t0 + t) < lens
        plsc.store_scatter(out_v, [idx], one16, mask=mask)
        return carry

    lax.fori_loop(0, T_PER_W, body, 0)
    pltpu.sync_copy(out_v, out_hbm.at[pl.ds(t0 * ROW, T_PER_W * ROW)])


def kernel(data, lengths, embed_init):
    del embed_init
    out2d = _onehot_sc(data, lengths)
    return out2d.reshape(MAX_LEN, BATCH, NUM_AA)


# speedup vs baseline: 3.2173x; 1.4086x over previous
import functools

import jax
import jax.numpy as jnp
from jax import lax
from jax.experimental import pallas as pl
from jax.experimental.pallas import tpu as pltpu
from jax.experimental.pallas import tpu_sc as plsc

MAX_LEN = 2048
BATCH = 16
NUM_AA = 21
NUM_CORES = 2
NUM_SUBCORES = 16
NUM_WORKERS = NUM_CORES * NUM_SUBCORES
T_PER_W = MAX_LEN // NUM_WORKERS

_mesh = plsc.VectorSubcoreMesh(core_axis_name="c", subcore_axis_name="s")


@functools.partial(
    pl.kernel,
    mesh=_mesh,
    out_type=jax.ShapeDtypeStruct((MAX_LEN, BATCH, NUM_AA), jnp.float32),
    scratch_types=[
        pltpu.VMEM((T_PER_W, BATCH), jnp.int32),
        pltpu.VMEM((BATCH,), jnp.int32),
        pltpu.VMEM((T_PER_W // 2, BATCH, NUM_AA), jnp.float32),
    ],
    compiler_params=pltpu.CompilerParams(needs_layout_passes=False),
)
def _onehot_sc(data_hbm, len_hbm, out_hbm, data_v, len_v, out_v):
    wid = lax.axis_index("s") * NUM_CORES + lax.axis_index("c")
    t0 = wid * T_PER_W
    pltpu.sync_copy(data_hbm.at[pl.ds(t0, T_PER_W)], data_v)
    pltpu.sync_copy(len_hbm, len_v)

    lanes = lax.iota(jnp.int32, 16)
    lens = len_v[...]
    zero16 = jnp.zeros((16,), jnp.float32)
    one16 = jnp.ones((16,), jnp.float32)
    half = T_PER_W // 2

    for c in range(2):
        def body(t, carry):
            for b in range(BATCH):
                out_v[t, b, pl.ds(0, 16)] = zero16
                out_v[t, b, pl.ds(NUM_AA - 16, 16)] = zero16
            tg = c * half + t
            row = data_v[tg]
            tvec = jnp.full((16,), t, jnp.int32)
            mask = (t0 + tg) < lens
            plsc.store_scatter(out_v, [tvec, lanes, row], one16, mask=mask)
            return carry

        lax.fori_loop(0, half, body, 0)
        pltpu.sync_copy(out_v, out_hbm.at[pl.ds(t0 + c * half, half)])


def kernel(data, lengths, embed_init):
    del embed_init
    return _onehot_sc(data, lengths)


# speedup vs baseline: 5.2586x; 1.6345x over previous
import functools

import jax
import jax.numpy as jnp
from jax import lax
from jax.experimental import pallas as pl
from jax.experimental.pallas import tpu as pltpu
from jax.experimental.pallas import tpu_sc as plsc

MAX_LEN = 2048
BATCH = 16
NUM_AA = 21
NUM_CORES = 2
NUM_SUBCORES = 16
B_HALF = BATCH // 2
T_CHUNK = 128
N_TCHUNK = MAX_LEN // T_CHUNK

_mesh = plsc.VectorSubcoreMesh(core_axis_name="c", subcore_axis_name="s")


@functools.partial(
    pl.kernel,
    mesh=_mesh,
    out_type=jax.ShapeDtypeStruct((NUM_AA, BATCH, MAX_LEN), jnp.float32),
    scratch_types=[
        pltpu.VMEM((T_CHUNK, BATCH), jnp.int32),
        pltpu.VMEM((BATCH,), jnp.int32),
        pltpu.VMEM((NUM_AA, B_HALF, T_CHUNK), jnp.float32),
    ],
    compiler_params=pltpu.CompilerParams(needs_layout_passes=False),
)
def _onehot_sc(data_hbm, len_hbm, out_hbm, data_v, len_v, out_v):
    wid = lax.axis_index("s") * NUM_CORES + lax.axis_index("c")
    h = wid % 2
    tc = wid // 2
    t1 = tc * T_CHUNK
    b0 = h * B_HALF
    pltpu.sync_copy(data_hbm.at[pl.ds(t1, T_CHUNK)], data_v)
    pltpu.sync_copy(len_hbm, len_v)

    lanes = lax.iota(jnp.int32, 16)
    lens = len_v[...]
    zero16 = jnp.zeros((16,), jnp.float32)
    one16 = jnp.ones((16,), jnp.float32)
    inhalf = (lanes >= b0) & (lanes < b0 + B_HALF)

    def zero_body(a, carry):
        for b in range(B_HALF):
            for j in range(T_CHUNK // 16):
                out_v[a, b, pl.ds(j * 16, 16)] = zero16
        return carry

    lax.fori_loop(0, NUM_AA, zero_body, 0)

    def body(t, carry):
        row = data_v[t]
        tvec = jnp.full((16,), t, jnp.int32)
        mask = inhalf & ((t1 + t) < lens)
        plsc.store_scatter(out_v, [row, lanes - b0, tvec], one16, mask=mask)
        return carry

    lax.fori_loop(0, T_CHUNK, body, 0)
    pltpu.sync_copy(out_v, out_hbm.at[:, pl.ds(b0, B_HALF), pl.ds(t1, T_CHUNK)])


def kernel(data, lengths, embed_init):
    del embed_init
    out = _onehot_sc(data, lengths)
    return jnp.transpose(out, (2, 1, 0))


# speedup vs baseline: 5.3332x; 1.0142x over previous
import functools

import jax
import jax.numpy as jnp
from jax import lax
from jax.experimental import pallas as pl
from jax.experimental.pallas import tpu as pltpu
from jax.experimental.pallas import tpu_sc as plsc

MAX_LEN = 2048
BATCH = 16
NUM_AA = 21
NUM_CORES = 2
NUM_SUBCORES = 16
B_HALF = BATCH // 2
T_CHUNK = 128
N_TCHUNK = MAX_LEN // T_CHUNK

_mesh = plsc.VectorSubcoreMesh(core_axis_name="c", subcore_axis_name="s")


@functools.partial(
    pl.kernel,
    mesh=_mesh,
    out_type=jax.ShapeDtypeStruct((NUM_AA, BATCH, MAX_LEN), jnp.float32),
    scratch_types=[
        pltpu.VMEM((T_CHUNK, BATCH), jnp.int32),
        pltpu.VMEM((BATCH,), jnp.int32),
        pltpu.VMEM((NUM_AA, B_HALF, T_CHUNK), jnp.float32),
        pltpu.SemaphoreType.DMA,
    ],
    compiler_params=pltpu.CompilerParams(needs_layout_passes=False),
)
def _onehot_sc(data_hbm, len_hbm, out_hbm, data_v, len_v, out_v, sem):
    wid = lax.axis_index("s") * NUM_CORES + lax.axis_index("c")
    h = wid % 2
    tc = wid // 2
    t1 = tc * T_CHUNK
    b0 = h * B_HALF
    cp_data = pltpu.async_copy(data_hbm.at[pl.ds(t1, T_CHUNK)], data_v, sem)
    cp_len = pltpu.async_copy(len_hbm, len_v, sem)

    lanes = lax.iota(jnp.int32, 16)
    zero16 = jnp.zeros((16,), jnp.float32)
    one16 = jnp.ones((16,), jnp.float32)

    def zero_body(a, carry):
        for b in range(B_HALF):
            for j in range(T_CHUNK // 16):
                out_v[a, b, pl.ds(j * 16, 16)] = zero16
        return carry

    lax.fori_loop(0, NUM_AA, zero_body, 0)
    cp_data.wait()
    cp_len.wait()

    bloc = lanes & 7
    bvec = bloc + b0
    sel8 = lanes >> 3
    lens_g = plsc.load_gather(len_v, [bvec])

    def body(i, carry):
        tvec = jnp.full((16,), 2 * i, jnp.int32) + sel8
        row = plsc.load_gather(data_v, [tvec, bvec])
        mask = (t1 + tvec) < lens_g
        plsc.store_scatter(out_v, [row, bloc, tvec], one16, mask=mask)
        return carry

    lax.fori_loop(0, T_CHUNK // 2, body, 0)
    pltpu.sync_copy(out_v, out_hbm.at[:, pl.ds(b0, B_HALF), pl.ds(t1, T_CHUNK)])


def kernel(data, lengths, embed_init):
    del embed_init
    out = _onehot_sc(data, lengths)
    return jnp.transpose(out, (2, 1, 0))
